# Initial kernel scaffold; baseline (speedup 1.0000x reference)
import jax
import jax.numpy as jnp
from jax.experimental import pallas as pl


def kernel(diagrams, grid):
    raise NotImplementedError("write your pallas kernel here")


# speedup vs baseline: 8.0398x; 8.0398x over previous
import functools

import jax
import jax.numpy as jnp
from jax import lax
from jax.experimental import pallas as pl
from jax.experimental.pallas import tpu as pltpu
from jax.experimental.pallas import tpu_sc as plsc

_L = 16
_U = 8


@functools.lru_cache(maxsize=None)
def _build(total_out: int, gx: int, gy: int):
    info = plsc.get_sparse_core_info()
    nw = info.num_cores * info.num_subcores
    assert total_out % nw == 0
    per_w_out = total_out // nw
    chunk_out = min(8192, per_w_out)
    assert per_w_out % chunk_out == 0
    n_chunks = per_w_out // chunk_out
    chunk_in = 2 * chunk_out
    n_vec = chunk_out // _L
    assert n_vec % _U == 0

    tbl_n = gx * gy
    sx = float(gx * gy * _L)
    mx = (gx - 1) * gy * _L
    sy = float(gy * _L)
    my = (gy - 1) * _L

    mesh = plsc.VectorSubcoreMesh(core_axis_name="c", subcore_axis_name="s")

    @functools.partial(
        pl.kernel,
        mesh=mesh,
        out_type=jax.ShapeDtypeStruct((total_out,), jnp.float32),
        compiler_params=pltpu.CompilerParams(needs_layout_passes=False),
        scratch_types=[
            pltpu.VMEM((chunk_in,), jnp.float32),
            pltpu.VMEM((chunk_in,), jnp.float32),
            pltpu.VMEM((chunk_out,), jnp.float32),
            pltpu.VMEM((chunk_out,), jnp.float32),
            pltpu.VMEM((tbl_n * _L,), jnp.float32),
            pltpu.SemaphoreType.DMA,
            pltpu.SemaphoreType.DMA,
            pltpu.SemaphoreType.DMA,
        ],
    )
    def grid_lookup(din_hbm, tbl_hbm, out_hbm, in0, in1, o0, o1, tblv,
                    sin, so0, so1):
        ins = (in0, in1)
        outs = (o0, o1)
        souts = (so0, so1)
        wid = lax.axis_index("s") * info.num_cores + lax.axis_index("c")
        in_base = wid * (2 * per_w_out)
        out_base = wid * per_w_out

        pltpu.sync_copy(tbl_hbm, tblv)

        iota = lax.iota(jnp.int32, _L)
        idx_e0 = iota * 2

        def start_in(c):
            return pltpu.async_copy(
                din_hbm.at[pl.ds(in_base + c * chunk_in, chunk_in)],
                ins[c % 2], sin)

        in_h = start_in(0)
        out_h = {}
        for c in range(n_chunks):
            in_h.wait()
            if c + 1 < n_chunks:
                in_h = start_in(c + 1)
            if c >= 2:
                out_h[c - 2].wait()
            inv = ins[c % 2]
            outv = outs[c % 2]

            @plsc.parallel_loop(0, n_vec, unroll=_U)
            def _(i):
                ie = idx_e0 + i * (2 * _L)
                io = ie + 1
                xs = plsc.load_gather(inv, [ie])
                ys = plsc.load_gather(inv, [io])
                fx = (xs * sx).astype(jnp.int32) & mx
                fy = (ys * sy).astype(jnp.int32) & my
                w = plsc.load_gather(tblv, [fx | fy | iota])
                outv[pl.ds(i * _L, _L)] = w

            out_h[c] = pltpu.async_copy(
                outv,
                out_hbm.at[pl.ds(out_base + c * chunk_out, chunk_out)],
                souts[c % 2])
        out_h[n_chunks - 2].wait()
        out_h[n_chunks - 1].wait()

    return grid_lookup


def kernel(diagrams, grid):
    b, n, _ = diagrams.shape
    gx, gy = grid.shape
    din = diagrams.reshape(-1)
    tbl_rep = jnp.broadcast_to(grid.reshape(-1)[:, None], (gx * gy, _L))
    out = _build(b * n, gx, gy)(din, tbl_rep.reshape(-1))
    return out.reshape(b, n)
